# Initial kernel scaffold; baseline (speedup 1.0000x reference)
import jax
import jax.numpy as jnp
from jax.experimental import pallas as pl


def kernel(x, edge_index, edge_attr, rev_map, batch, W_i, W_m, W_a, b_a, bn_gamma, bn_beta, R1_w, R1_b, R2_w, R2_b):
    raise NotImplementedError("write your pallas kernel here")


# speedup vs baseline: 1.0040x; 1.0040x over previous
import functools

import jax
import jax.numpy as jnp
from jax import lax
from jax.experimental import pallas as pl
from jax.experimental.pallas import tpu as pltpu
from jax.experimental.pallas import tpu_sc as plsc

F32 = jnp.float32
I32 = jnp.int32

N_NODE = 10000
N_EDGE = 320000
D_V = 128
D_E = 16
D_H = 256
N_GRAPH = 256
T_STEPS = 5
HALF = 128

C = 128
SLAB = 8
GRP = 2
SCH = N_EDGE // C
SCH_PAD = 2520
NROWS_TILE = 624

_sds = jax.ShapeDtypeStruct


def _mesh():
    return plsc.VectorSubcoreMesh(core_axis_name="c", subcore_axis_name="s")


def _slab_params(s):
    rbase = jnp.where(s < 15, 160 * s, 2400)
    nslab = jnp.where(s < 15, 20, 12)
    return rbase, nslab


def _zero_node_table(zer_ref, u_sh, s):
    pltpu.sync_copy(zer_ref, u_sh.at[pl.ds(s * NROWS_TILE, NROWS_TILE)])

    @pl.when(s == 15)
    def _():
        pltpu.sync_copy(zer_ref.at[pl.ds(0, 16)],
                        u_sh.at[pl.ds(16 * NROWS_TILE, 16)])


def _dump_node_table(u_sh, out_ref, s):
    pltpu.sync_copy(u_sh.at[pl.ds(s * NROWS_TILE, NROWS_TILE)],
                    out_ref.at[pl.ds(s * NROWS_TILE, NROWS_TILE)])

    @pl.when(s == 15)
    def _():
        pltpu.sync_copy(u_sh.at[pl.ds(16 * NROWS_TILE, 16)],
                        out_ref.at[pl.ds(16 * NROWS_TILE, 16)])


def _gather_slab(tab_ref, idx_sl, out_ref, buf, sem, r0, rows):
    for g in range(rows // GRP):
        cps = [
            pltpu.async_copy(tab_ref.at[idx_sl.at[GRP * g + j]],
                             buf.at[pl.ds(j * C, C)], sem)
            for j in range(GRP)
        ]
        for cp in cps:
            cp.wait()
        pltpu.sync_copy(buf, out_ref.at[pl.ds((r0 + GRP * g) * C, GRP * C)])


def _gather_half(tab_ref, src2_ref, out_ref, idx_sl, buf, sem, s):
    rbase, nslab = _slab_params(s)

    def body(k, carry):
        r0 = pl.multiple_of(rbase + SLAB * k, 8)

        @pl.when(k < nslab)
        def _():
            pltpu.sync_copy(src2_ref.at[pl.ds(r0, SLAB)], idx_sl)
            _gather_slab(tab_ref, idx_sl, out_ref, buf, sem, r0, SLAB)

        return carry

    lax.fori_loop(0, 20, body, 0)

    @pl.when(s == 15)
    def _():
        pltpu.sync_copy(src2_ref.at[pl.ds(2496, SLAB)], idx_sl)
        _gather_slab(tab_ref, idx_sl, out_ref, buf, sem, 2496, 4)


CH = 128
SCAN_PAD = N_EDGE + 128


def _shard_start(j):
    half = j // 16
    r = j - 16 * half
    st = jnp.where(r <= 5, 10080 * r, 50400 + 9968 * (r - 5))
    return half * 160000 + st


def _shard_size(j):
    r = j % 16
    return jnp.where(r < 5, 10080, jnp.where(r < 15, 9968, 9920))


def _scan_chunk(tab_ref, rpi_ref, flg_ref, scan_ref, idxb, flgb, rowb, sem,
                base, cnt, acc):
    pltpu.sync_copy(rpi_ref.at[pl.ds(base, cnt)], idxb.at[pl.ds(0, cnt)])
    pltpu.async_copy(tab_ref.at[idxb.at[pl.ds(0, cnt)]],
                     rowb.at[pl.ds(0, cnt)], sem).wait()
    pltpu.sync_copy(flg_ref.at[pl.ds(base, cnt)], flgb.at[pl.ds(0, cnt)])

    def row(i, acc8):
        m = flgb[i, pl.ds(0, 16)]
        new = tuple(rowb[i, pl.ds(16 * q, 16)] + acc8[q] * m for q in range(8))
        for q in range(8):
            rowb[i, pl.ds(16 * q, 16)] = new[q]
        return new

    acc = lax.fori_loop(0, cnt, row, acc)
    pltpu.sync_copy(rowb.at[pl.ds(0, cnt)], scan_ref.at[pl.ds(base, cnt)])
    return acc


def _scan_half(tab_ref, rpi_ref, flg_ref, zer_ref, scan_ref,
               idxb, flgb, rowb, sem, s):
    @pl.when(s == 0)
    def _():
        pltpu.sync_copy(zer_ref.at[pl.ds(0, 128)],
                        scan_ref.at[pl.ds(N_EDGE, 128)])

    for k in range(2):
        j = 2 * s + k
        st = pl.multiple_of(_shard_start(j), 8)
        sz = _shard_size(j)
        nfull = sz // CH
        tail = sz - nfull * CH

        zeros8 = tuple(jnp.zeros((16,), F32) for _ in range(8))

        def chunk(i, acc):
            base = pl.multiple_of(st + i * CH, 8)
            return _scan_chunk(tab_ref, rpi_ref, flg_ref, scan_ref,
                               idxb, flgb, rowb, sem, base, CH, acc)

        acc = lax.fori_loop(0, nfull, chunk, zeros8)
        tbase = pl.multiple_of(st + nfull * CH, 8)
        for tsz in (96, 112, 64):
            @pl.when(tail == tsz)
            def _(tbase=tbase, tsz=tsz, acc=acc):
                _scan_chunk(tab_ref, rpi_ref, flg_ref, scan_ref,
                            idxb, flgb, rowb, sem, tbase, tsz, acc)


def _ubuild_half(scan_ref, p1b, p2b, u_ref, bufA, bufB, sem, s):
    for k in range(5):
        j = s * 5 + k
        pltpu.async_copy(scan_ref.at[p1b.at[j]], bufA, sem).wait()
        pltpu.async_copy(scan_ref.at[p2b.at[j]], bufB, sem).wait()

        def row(i, carry):
            for q in range(8):
                bufA[i, pl.ds(16 * q, 16)] = (bufA[i, pl.ds(16 * q, 16)]
                                              + bufB[i, pl.ds(16 * q, 16)])
            return carry

        lax.fori_loop(0, 128, row, 0)
        pltpu.sync_copy(bufA, u_ref.at[pl.ds(j * 128, 128)])


NP_PAD = 10240
G_TAB = 264


def _nsegsum_half(hv_ref, bat2_ref, zer_ref, out_ref, g_sh, idx_loc, buf, sem, s):
    pltpu.sync_copy(bat2_ref, idx_loc)

    @pl.when(s == 0)
    def _():
        pltpu.sync_copy(zer_ref.at[pl.ds(0, G_TAB)], g_sh)
    plsc.subcore_barrier()

    for j in range(5):
        pltpu.sync_copy(hv_ref.at[pl.ds((s * 5 + j) * C, C)], buf)
        pltpu.sync_copy(buf, g_sh.at[idx_loc.at[s * 5 + j]], add=True)

    plsc.subcore_barrier()

    @pl.when(s == 0)
    def _():
        pltpu.sync_copy(g_sh.at[pl.ds(0, N_GRAPH)], out_ref)


@functools.partial(
    pl.kernel,
    out_type=(_sds((N_EDGE, HALF), F32), _sds((N_EDGE, HALF), F32)),
    mesh=_mesh(),
    scratch_types=[
        pltpu.VMEM((SLAB, C), I32),
        pltpu.VMEM((GRP * C, HALF), F32),
        pltpu.SemaphoreType.DMA,
    ],
)
def _sc_gather(y_lo, y_hi, src2, g_lo, g_hi, idx_sl, buf, sem):
    c = lax.axis_index("c")
    s = lax.axis_index("s")

    @pl.when(c == 0)
    def _():
        _gather_half(y_lo, src2, g_lo, idx_sl, buf, sem, s)

    @pl.when(c == 1)
    def _():
        _gather_half(y_hi, src2, g_hi, idx_sl, buf, sem, s)


@functools.partial(
    pl.kernel,
    out_type=(_sds((SCAN_PAD, HALF), F32), _sds((SCAN_PAD, HALF), F32)),
    mesh=_mesh(),
    scratch_types=[
        pltpu.VMEM((CH,), I32),
        pltpu.VMEM((CH, 16), F32),
        pltpu.VMEM((CH, HALF), F32),
        pltpu.SemaphoreType.DMA,
    ],
)
def _sc_scan(h_lo, h_hi, rpi, flg, zer, scan_lo, scan_hi, idxb, flgb, rowb, sem):
    c = lax.axis_index("c")
    s = lax.axis_index("s")

    @pl.when(c == 0)
    def _():
        _scan_half(h_lo, rpi, flg, zer, scan_lo, idxb, flgb, rowb, sem, s)

    @pl.when(c == 1)
    def _():
        _scan_half(h_hi, rpi, flg, zer, scan_hi, idxb, flgb, rowb, sem, s)


@functools.partial(
    pl.kernel,
    out_type=(_sds((NP_PAD, HALF), F32), _sds((NP_PAD, HALF), F32)),
    mesh=_mesh(),
    scratch_types=[
        pltpu.VMEM((NP_PAD // C, C), I32),
        pltpu.VMEM((NP_PAD // C, C), I32),
        pltpu.VMEM((C, HALF), F32),
        pltpu.VMEM((C, HALF), F32),
        pltpu.SemaphoreType.DMA,
    ],
)
def _sc_ubuild(scan_lo, scan_hi, pos1, pos2, u_lo, u_hi, p1b, p2b, bufA, bufB, sem):
    c = lax.axis_index("c")
    s = lax.axis_index("s")
    pltpu.sync_copy(pos1, p1b)
    pltpu.sync_copy(pos2, p2b)

    @pl.when(c == 0)
    def _():
        _ubuild_half(scan_lo, p1b, p2b, u_lo, bufA, bufB, sem, s)

    @pl.when(c == 1)
    def _():
        _ubuild_half(scan_hi, p1b, p2b, u_hi, bufA, bufB, sem, s)


@functools.partial(
    pl.kernel,
    out_type=(_sds((N_GRAPH, HALF), F32), _sds((N_GRAPH, HALF), F32)),
    mesh=_mesh(),
    scratch_types=[
        pltpu.VMEM_SHARED((G_TAB, HALF), F32),
        pltpu.VMEM((NP_PAD // C, C), I32),
        pltpu.VMEM((C, HALF), F32),
        pltpu.SemaphoreType.DMA,
    ],
)
def _sc_nsegsum(hv_lo, hv_hi, bat2, zer, hg_lo, hg_hi, g_sh, idx_loc, buf, sem):
    c = lax.axis_index("c")
    s = lax.axis_index("s")

    @pl.when(c == 0)
    def _():
        _nsegsum_half(hv_lo, bat2, zer, hg_lo, g_sh, idx_loc, buf, sem, s)

    @pl.when(c == 1)
    def _():
        _nsegsum_half(hv_hi, bat2, zer, hg_hi, g_sh, idx_loc, buf, sem, s)


def _mm(a, b_t):
    return lax.dot_general(a, b_t, (((1,), (1,)), ((), ())),
                           preferred_element_type=F32)


BN = 400
BE = 512


def _tc_edge_init_body(xg_ref, ea_ref, wi_ref,
                       h0l_ref, h0h_ref, hl_ref, hh_ref):
    h0 = _mm(jnp.concatenate([xg_ref[...], ea_ref[...]], axis=1), wi_ref[...])
    h = jax.nn.relu(h0)
    h0l_ref[...] = h0[:, :HALF]
    h0h_ref[...] = h0[:, HALF:]
    hl_ref[...] = h[:, :HALF]
    hh_ref[...] = h[:, HALF:]


def _tc_update_body(h0l_ref, h0h_ref, gl_ref, gh_ref, hrl_ref, hrh_ref, wm_ref,
                    hl_ref, hh_ref):
    msg = jnp.concatenate([gl_ref[...] - hrl_ref[...],
                           gh_ref[...] - hrh_ref[...]], axis=1)
    h = jax.nn.relu(jnp.concatenate([h0l_ref[...], h0h_ref[...]], axis=1)
                    + _mm(msg, wm_ref[...]))
    hl_ref[...] = h[:, :HALF]
    hh_ref[...] = h[:, HALF:]


def _tc_node_body(x_ref, mvl_ref, mvh_ref, wax_ref, wam_ref, ba_ref,
                  hvl_ref, hvh_ref):
    mv = jnp.concatenate([mvl_ref[...], mvh_ref[...]], axis=1)
    hv = jax.nn.relu(_mm(x_ref[...], wax_ref[...]) + _mm(mv, wam_ref[...])
                     + ba_ref[...])
    hvl_ref[...] = hv[:, :HALF]
    hvh_ref[...] = hv[:, HALF:]


def _tc_head_body(hgl_ref, hgh_ref, gam_ref, bet_ref, r1w_ref, r1b_ref,
                  r2w_ref, r2b_ref, out_ref):
    hg = jnp.concatenate([hgl_ref[...], hgh_ref[...]], axis=1)
    mean = jnp.mean(hg, axis=0, keepdims=True)
    var = jnp.mean((hg - mean) * (hg - mean), axis=0, keepdims=True)
    hgn = (hg - mean) / jnp.sqrt(var + 1e-5) * gam_ref[...] + bet_ref[...]
    hid = jax.nn.relu(_mm(hgn, r1w_ref[...]) + r1b_ref[...])
    out_ref[...] = _mm(hid, r2w_ref[...]) + r2b_ref[0, 0]


def _row_spec(block_rows, width):
    return pl.BlockSpec((block_rows, width), lambda i: (i, 0))


def _const_spec(shape):
    return pl.BlockSpec(shape, lambda i: (0,) * len(shape))


def kernel(x, edge_index, edge_attr, rev_map, batch, W_i, W_m, W_a, b_a,
           bn_gamma, bn_beta, R1_w, R1_b, R2_w, R2_b):
    src = edge_index[0].astype(I32)
    rev = rev_map.astype(I32)
    idx_pad = ((0, SCH_PAD - SCH), (0, 0))
    rev2 = jnp.pad(rev.reshape(SCH, C), idx_pad)
    src2 = jnp.pad(src.reshape(SCH, C), idx_pad)
    zer = jnp.zeros((NROWS_TILE, HALF), F32)
    bat2 = jnp.concatenate(
        [batch.astype(I32), jnp.full((NP_PAD - N_NODE,), N_GRAPH, I32)]
    ).reshape(NP_PAD // C, C)

    pi = jnp.argsort(src).astype(I32)
    srcs = src[pi]
    rpi = rev[pi]
    iota = jnp.arange(N_EDGE, dtype=I32)
    shard_starts = jnp.asarray(
        [160000 * h + (10080 * r if r <= 5 else 50400 + 9968 * (r - 5))
         for h in range(2) for r in range(16)], I32)
    chg = jnp.concatenate([jnp.zeros((1,), F32),
                           (srcs[1:] == srcs[:-1]).astype(F32)])
    flg = jnp.broadcast_to(chg.at[shard_starts].set(0.0)[:, None],
                           (N_EDGE, 16)) + jnp.zeros((N_EDGE, 16), F32)
    DUMMY = N_EDGE
    last = jnp.full((N_NODE,), -1, I32).at[srcs].max(iota)
    first = jnp.full((N_NODE,), N_EDGE, I32).at[srcs].min(iota)
    bnd = jnp.concatenate([shard_starts[1:], jnp.asarray([N_EDGE + 1], I32)])
    bnext = bnd[jnp.searchsorted(shard_starts[1:], first, side="right")
                .astype(I32)]
    pos1 = jnp.where(last >= 0, last, DUMMY)
    pos2 = jnp.where((bnext <= last) & (last >= 0), bnext - 1, DUMMY)
    pos_pad = (0, NP_PAD - N_NODE)
    pos1_2 = jnp.pad(pos1, pos_pad, constant_values=DUMMY).reshape(NP_PAD // C, C)
    pos2_2 = jnp.pad(pos2, pos_pad, constant_values=DUMMY).reshape(NP_PAD // C, C)

    w_ax = W_a[:, :D_V]
    w_am = W_a[:, D_V:]
    ba = b_a.reshape(1, D_H)
    gam = bn_gamma.reshape(1, D_H)
    bet = bn_beta.reshape(1, D_H)
    r1b = R1_b.reshape(1, D_H)
    r2w = jnp.zeros((HALF, D_H), F32).at[:1].set(R2_w)
    r2b = R2_b.reshape(1, 1)

    ng = N_NODE // BN
    eg = N_EDGE // BE

    xg, _ = _sc_gather(x, x, src2)

    h0_lo, h0_hi, h_lo, h_hi = pl.pallas_call(
        _tc_edge_init_body,
        grid=(eg,),
        in_specs=[_row_spec(BE, D_V), _row_spec(BE, D_E),
                  _const_spec((D_H, D_V + D_E))],
        out_specs=[_row_spec(BE, HALF)] * 4,
        out_shape=[_sds((N_EDGE, HALF), F32)] * 4,
    )(xg, edge_attr, W_i)

    def seg_sum(a_lo, a_hi, idx2):
        scan_lo, scan_hi = _sc_scan(a_lo, a_hi, idx2, flg, zer)
        return _sc_ubuild(scan_lo, scan_hi, pos1_2, pos2_2)

    for t in range(T_STEPS):
        u_lo, u_hi = seg_sum(h_lo, h_hi, rpi)
        g_lo, g_hi = _sc_gather(u_lo, u_hi, src2)
        hr_lo, hr_hi = _sc_gather(h_lo, h_hi, rev2)
        h_lo, h_hi = pl.pallas_call(
            _tc_update_body,
            grid=(eg,),
            in_specs=[_row_spec(BE, HALF)] * 6 + [_const_spec((D_H, D_H))],
            out_specs=[_row_spec(BE, HALF)] * 2,
            out_shape=[_sds((N_EDGE, HALF), F32)] * 2,
        )(h0_lo, h0_hi, g_lo, g_hi, hr_lo, hr_hi, W_m)

    mv_lo, mv_hi = seg_sum(h_lo, h_hi, pi)

    hv_lo, hv_hi = pl.pallas_call(
        _tc_node_body,
        grid=(ng,),
        in_specs=[_row_spec(BN, D_V), _row_spec(BN, HALF), _row_spec(BN, HALF),
                  _const_spec((D_H, D_V)), _const_spec((D_H, D_H)),
                  _const_spec((1, D_H))],
        out_specs=[_row_spec(BN, HALF)] * 2,
        out_shape=[_sds((N_NODE, HALF), F32)] * 2,
    )(x, mv_lo[:N_NODE], mv_hi[:N_NODE], w_ax, w_am, ba)

    pad = ((0, NP_PAD - N_NODE), (0, 0))
    hv_lo_p = jnp.pad(hv_lo, pad)
    hv_hi_p = jnp.pad(hv_hi, pad)

    hg_lo, hg_hi = _sc_nsegsum(hv_lo_p, hv_hi_p, bat2, zer)

    out = pl.pallas_call(
        _tc_head_body,
        grid=(1,),
        in_specs=[_const_spec((N_GRAPH, HALF)), _const_spec((N_GRAPH, HALF)),
                  _const_spec((1, D_H)), _const_spec((1, D_H)),
                  _const_spec((D_H, D_H)), _const_spec((1, D_H)),
                  _const_spec((HALF, D_H)), _const_spec((1, 1))],
        out_specs=_const_spec((N_GRAPH, HALF)),
        out_shape=_sds((N_GRAPH, HALF), F32),
    )(hg_lo, hg_hi, gam, bet, R1_w, r1b, r2w, r2b)

    return out[:, :1]
